# baseline (speedup 1.0000x reference)
import functools

import jax
import jax.numpy as jnp
from jax import lax
from jax.experimental import pallas as pl
from jax.experimental.pallas import tpu as pltpu
from jax.experimental.pallas import tpu_sc as plsc

NC = 2
NS = 16
L = 16
ROWS = 8


def kernel(x, pe):
    batch, seq_len, d_model = x.shape
    nw = NC * NS
    seq_per_w = seq_len // nw
    n_steps = seq_per_w // ROWS
    nj = d_model // L
    mesh = plsc.VectorSubcoreMesh(core_axis_name="c", subcore_axis_name="s")

    @functools.partial(
        pl.kernel,
        mesh=mesh,
        out_type=jax.ShapeDtypeStruct((batch, seq_len, d_model), x.dtype),
        scratch_types=[
            pltpu.VMEM((batch, ROWS, d_model), jnp.float32),
            pltpu.VMEM((2, ROWS, d_model), jnp.float32),
            [pltpu.SemaphoreType.DMA] * batch,
            [pltpu.SemaphoreType.DMA] * batch,
            [pltpu.SemaphoreType.DMA] * 2,
        ],
    )
    def sc_add(x_hbm, pe_hbm, out_hbm, x_bufs, pe_bufs, sx, so, spe):
        wid = lax.axis_index("s") * NC + lax.axis_index("c")
        base = wid * seq_per_w

        def x_copy(i, b):
            rows = pl.ds(base + i * ROWS, ROWS)
            return pltpu.make_async_copy(x_hbm.at[b, rows, :], x_bufs.at[b], sx[b])

        def out_copy(i, b):
            rows = pl.ds(base + i * ROWS, ROWS)
            return pltpu.make_async_copy(x_bufs.at[b], out_hbm.at[b, rows, :], so[b])

        def pe_copy(i, ph):
            rows = pl.ds(base + i * ROWS, ROWS)
            return pltpu.make_async_copy(pe_hbm.at[rows, :], pe_bufs.at[ph], spe[ph])

        def do_block(i, b, ph):
            if b < batch - 2:
                @pl.when(i > 0)
                def _():
                    out_copy(i - 1, b + 2).wait()
                x_copy(i, b + 2).start()
            else:
                @pl.when(i < n_steps - 1)
                def _():
                    out_copy(i, b - 2).wait()
                    x_copy(i + 1, b - 2).start()
            x_copy(i, b).wait()

            def row_loop(r, c):
                for j in range(nj):
                    sl = pl.ds(j * L, L)
                    plsc.addupdate(x_bufs.at[b, r, sl], pe_bufs[ph, r, sl])
                return c

            lax.fori_loop(0, ROWS, row_loop, 0)
            out_copy(i, b).start()

        def pair(k, c):
            for ph in range(2):
                i = k * 2 + ph

                @pl.when(i + 1 < n_steps)
                def _():
                    pe_copy(i + 1, 1 - ph).start()

                pe_copy(i, ph).wait()
                for b in range(batch):
                    do_block(i, b, ph)
            return c

        pe_copy(0, 0).start()
        x_copy(0, 0).start()
        x_copy(0, 1).start()
        lax.fori_loop(0, n_steps // 2, pair, 0)
        for b in (batch - 2, batch - 1):
            out_copy(n_steps - 1, b).wait()

    return sc_add(x, pe[:seq_len])


# speedup vs baseline: 1.0743x; 1.0743x over previous
import functools

import jax
import jax.numpy as jnp
from jax import lax
from jax.experimental import pallas as pl
from jax.experimental.pallas import tpu as pltpu
from jax.experimental.pallas import tpu_sc as plsc

NC = 2
NS = 16
L = 16
ROWS = 8


def kernel(x, pe):
    batch, seq_len, d_model = x.shape
    nw = NC * NS
    seq_per_w = seq_len // nw
    n_steps = seq_per_w // ROWS
    nj = d_model // L
    mesh = plsc.VectorSubcoreMesh(core_axis_name="c", subcore_axis_name="s")

    @functools.partial(
        pl.kernel,
        mesh=mesh,
        out_type=jax.ShapeDtypeStruct((batch, seq_len, d_model), x.dtype),
        scratch_types=[
            pltpu.VMEM((2, batch, ROWS, d_model), jnp.float32),
            pltpu.VMEM((2, ROWS, d_model), jnp.float32),
            [pltpu.SemaphoreType.DMA] * (2 * batch),
            [pltpu.SemaphoreType.DMA] * (2 * batch),
            [pltpu.SemaphoreType.DMA] * 2,
        ],
    )
    def sc_add(x_hbm, pe_hbm, out_hbm, x_bufs, pe_bufs, sx, so, spe):
        wid = lax.axis_index("s") * NC + lax.axis_index("c")
        base = wid * seq_per_w

        def x_copy(i, ph, b):
            rows = pl.ds(base + i * ROWS, ROWS)
            return pltpu.make_async_copy(
                x_hbm.at[b, rows, :], x_bufs.at[ph, b], sx[ph * batch + b])

        def out_copy(i, ph, b):
            rows = pl.ds(base + i * ROWS, ROWS)
            return pltpu.make_async_copy(
                x_bufs.at[ph, b], out_hbm.at[b, rows, :], so[ph * batch + b])

        def pe_copy(i, ph):
            rows = pl.ds(base + i * ROWS, ROWS)
            return pltpu.make_async_copy(pe_hbm.at[rows, :], pe_bufs.at[ph], spe[ph])

        def do_block(i, b, ph):
            @pl.when(i > 0)
            def _():
                out_copy(i - 1, 1 - ph, b).wait()

            @pl.when(i < n_steps - 1)
            def _():
                x_copy(i + 1, 1 - ph, b).start()

            x_copy(i, ph, b).wait()

            def row_loop(r, c):
                for j in range(nj):
                    sl = pl.ds(j * L, L)
                    plsc.addupdate(x_bufs.at[ph, b, r, sl], pe_bufs[ph, r, sl])
                return c

            lax.fori_loop(0, ROWS, row_loop, 0)
            out_copy(i, ph, b).start()

        def pair(k, c):
            for ph in range(2):
                i = k * 2 + ph

                @pl.when(i + 1 < n_steps)
                def _():
                    pe_copy(i + 1, 1 - ph).start()

                pe_copy(i, ph).wait()
                for b in range(batch):
                    do_block(i, b, ph)
            return c

        pe_copy(0, 0).start()
        for b in range(batch):
            x_copy(0, 0, b).start()
        lax.fori_loop(0, n_steps // 2, pair, 0)
        for b in range(batch):
            out_copy(n_steps - 1, (n_steps - 1) % 2, b).wait()

    return sc_add(x, pe[:seq_len])


# speedup vs baseline: 1.0792x; 1.0045x over previous
import functools

import jax
import jax.numpy as jnp
from jax import lax
from jax.experimental import pallas as pl
from jax.experimental.pallas import tpu as pltpu
from jax.experimental.pallas import tpu_sc as plsc

NC = 2
NS = 16
L = 16
ROWS = 8


def kernel(x, pe):
    batch, seq_len, d_model = x.shape
    nw = NC * NS
    seq_per_w = seq_len // nw
    n_steps = seq_per_w // ROWS
    nj = d_model // L
    mesh = plsc.VectorSubcoreMesh(core_axis_name="c", subcore_axis_name="s")

    @functools.partial(
        pl.kernel,
        mesh=mesh,
        out_type=jax.ShapeDtypeStruct((batch, seq_len, d_model), x.dtype),
        scratch_types=[
            pltpu.VMEM((2, batch, ROWS, d_model), jnp.float32),
            pltpu.VMEM((2, ROWS, d_model), jnp.float32),
            [pltpu.SemaphoreType.DMA] * (2 * batch),
            [pltpu.SemaphoreType.DMA] * (2 * batch),
            [pltpu.SemaphoreType.DMA] * 2,
        ],
    )
    def sc_add(x_hbm, pe_hbm, out_hbm, x_bufs, pe_bufs, sx, so, spe):
        wid = lax.axis_index("s") * NC + lax.axis_index("c")
        base = wid * ROWS

        def x_copy(i, ph, b):
            rows = pl.ds(base + i * (nw * ROWS), ROWS)
            return pltpu.make_async_copy(
                x_hbm.at[b, rows, :], x_bufs.at[ph, b], sx[ph * batch + b])

        def out_copy(i, ph, b):
            rows = pl.ds(base + i * (nw * ROWS), ROWS)
            return pltpu.make_async_copy(
                x_bufs.at[ph, b], out_hbm.at[b, rows, :], so[ph * batch + b])

        def pe_copy(i, ph):
            rows = pl.ds(base + i * (nw * ROWS), ROWS)
            return pltpu.make_async_copy(pe_hbm.at[rows, :], pe_bufs.at[ph], spe[ph])

        def do_block(i, b, ph):
            @pl.when(i > 0)
            def _():
                out_copy(i - 1, 1 - ph, b).wait()

            @pl.when(i < n_steps - 1)
            def _():
                x_copy(i + 1, 1 - ph, b).start()

            x_copy(i, ph, b).wait()

            def row_loop(r, c):
                for j in range(nj):
                    sl = pl.ds(j * L, L)
                    plsc.addupdate(x_bufs.at[ph, b, r, sl], pe_bufs[ph, r, sl])
                return c

            lax.fori_loop(0, ROWS, row_loop, 0)
            out_copy(i, ph, b).start()

        def pair(k, c):
            for ph in range(2):
                i = k * 2 + ph

                @pl.when(i + 1 < n_steps)
                def _():
                    pe_copy(i + 1, 1 - ph).start()

                pe_copy(i, ph).wait()
                for b in range(batch):
                    do_block(i, b, ph)
            return c

        pe_copy(0, 0).start()
        for b in range(batch):
            x_copy(0, 0, b).start()
        lax.fori_loop(0, n_steps // 2, pair, 0)
        for b in range(batch):
            out_copy(n_steps - 1, (n_steps - 1) % 2, b).wait()

    return sc_add(x, pe[:seq_len])
